# Initial kernel scaffold; baseline (speedup 1.0000x reference)
import jax
import jax.numpy as jnp
from jax.experimental import pallas as pl


def kernel(x1, edge_index1, x2, edge_index2, W1a, b1a, W1b, b1b, W2a, b2a, W2b, b2b, Wd, bd, Wo, bo):
    raise NotImplementedError("write your pallas kernel here")


# speedup vs baseline: 11.3604x; 11.3604x over previous
import functools

import jax
import jax.numpy as jnp
from jax import lax
from jax.experimental import pallas as pl
from jax.experimental.pallas import tpu as pltpu
from jax.experimental.pallas import tpu_sc as plsc

_N = 10000
_E = 320000
_D = 128
_H = 64
_HH = _H // 2
_H4 = _H // 4

_NC = 2
_NS = 16
_K = 100
_NCH = _E // (_NS * _K)
_NP = 10240
_ZR = _NP // _NS
_G = 8

_NB1 = 5
_RB1 = _N // _NB1
_NB3 = 10
_RB3 = _NP // _NB3


def _stage1(x_cat, Wa, ba):
  def body(x_ref, w_ref, b_ref, o_ref):
    o_ref[...] = jnp.maximum(
        jnp.dot(x_ref[...], w_ref[0], preferred_element_type=jnp.float32)
        + b_ref[0], 0.0)

  return pl.pallas_call(
      body,
      grid=(2, _NB1),
      in_specs=[
          pl.BlockSpec((_RB1, _D), lambda g, j: (g * _NB1 + j, 0)),
          pl.BlockSpec((1, _D, _H), lambda g, j: (g, 0, 0)),
          pl.BlockSpec((1, 1, _H), lambda g, j: (g, 0, 0)),
      ],
      out_specs=pl.BlockSpec((_RB1, _H), lambda g, j: (g * _NB1 + j, 0)),
      out_shape=jax.ShapeDtypeStruct((2 * _N, _H), jnp.float32),
  )(x_cat, Wa, ba)


def _sc_scatter(h_cat, src_off, src_raw, dst_raw, z_h, z_c, ones_g):
  mesh = plsc.VectorSubcoreMesh(core_axis_name="c", subcore_axis_name="s")

  @functools.partial(
      pl.kernel,
      out_type=[
          jax.ShapeDtypeStruct((2 * _NP, _H), jnp.float32),
          jax.ShapeDtypeStruct((2 * _NP, _G), jnp.float32),
      ],
      mesh=mesh,
      scratch_types=[
          pltpu.VMEM((_NCH, _K), jnp.int32),
          pltpu.VMEM((_NCH, _K), jnp.int32),
          pltpu.VMEM((_NCH, _K), jnp.int32),
          pltpu.VMEM((_K, _H), jnp.float32),
          pltpu.VMEM((_K, _G), jnp.float32),
          pltpu.VMEM_SHARED((_NP, _H), jnp.float32),
          pltpu.VMEM_SHARED((_NP, _G), jnp.float32),
          pltpu.SemaphoreType.DMA,
      ],
      compiler_params=pltpu.CompilerParams(use_tc_tiling_on_sc=False),
  )
  def k(h_hbm, so_hbm, sr_hbm, dr_hbm, zh_hbm, zc_hbm, ones_hbm,
        a_out, c_out, so_v, sr_v, dr_v, msg_v, ones_v, a_s, c_s, sem):
    cid = lax.axis_index("c")
    sid = lax.axis_index("s")
    wid = cid * _NS + sid
    pltpu.sync_copy(zh_hbm, a_s.at[pl.ds(sid * _ZR, _ZR)])
    pltpu.sync_copy(zc_hbm, c_s.at[pl.ds(sid * _ZR, _ZR)])
    pltpu.sync_copy(ones_hbm, ones_v)
    row0 = wid * _NCH
    pltpu.sync_copy(so_hbm.at[pl.ds(row0, _NCH)], so_v)
    pltpu.sync_copy(sr_hbm.at[pl.ds(row0, _NCH)], sr_v)
    pltpu.sync_copy(dr_hbm.at[pl.ds(row0, _NCH)], dr_v)
    plsc.subcore_barrier()

    @pl.loop(0, _NCH)
    def _(j):
      pltpu.async_copy(h_hbm.at[so_v.at[j]], msg_v, sem).wait()
      pltpu.sync_copy(msg_v, a_s.at[dr_v.at[j]], add=True)
      pltpu.sync_copy(ones_v, c_s.at[sr_v.at[j]], add=True)

    plsc.subcore_barrier()
    base = cid * _NP + sid * _ZR
    pltpu.sync_copy(a_s.at[pl.ds(sid * _ZR, _ZR)], a_out.at[pl.ds(base, _ZR)])
    pltpu.sync_copy(c_s.at[pl.ds(sid * _ZR, _ZR)], c_out.at[pl.ds(base, _ZR)])

  return k(h_cat, src_off, src_raw, dst_raw, z_h, z_c, ones_g)


def _stage3(a_cat, c_cat, Wb, bb, Wd, bd, Wo, bo):
  def body(a_ref, c_ref, wb, bbr, wd, bdr, wo, bor, o_ref, acc0, acc1):
    g = pl.program_id(0)
    j = pl.program_id(1)
    y = jnp.dot(a_ref[...], wb[0], preferred_element_type=jnp.float32) + bbr[0]
    y = jnp.maximum(y, 0.0)
    part = jnp.sum(y * c_ref[:, 0:1], axis=0, keepdims=True)

    @pl.when((g == 0) & (j == 0))
    def _():
      acc0[...] = part

    @pl.when((g == 0) & (j > 0))
    def _():
      acc0[...] = acc0[...] + part

    @pl.when((g == 1) & (j == 0))
    def _():
      acc1[...] = part

    @pl.when((g == 1) & (j > 0))
    def _():
      acc1[...] = acc1[...] + part

    @pl.when((g == 1) & (j == _NB3 - 1))
    def _():
      p0 = acc0[...] * (1.0 / _N)
      p1 = acc1[...] * (1.0 / _N)
      t = (jnp.dot(p0, wd[0:_HH], preferred_element_type=jnp.float32)
           + jnp.dot(p1, wd[_HH:_H], preferred_element_type=jnp.float32)
           + bdr[...])
      t = jnp.maximum(t, 0.0)
      z = jnp.dot(t, wo[...], preferred_element_type=jnp.float32) + bor[...]
      o_ref[...] = 1.0 / (1.0 + jnp.exp(-z))

  return pl.pallas_call(
      body,
      grid=(2, _NB3),
      in_specs=[
          pl.BlockSpec((_RB3, _H), lambda g, j: (g * _NB3 + j, 0)),
          pl.BlockSpec((_RB3, _G), lambda g, j: (g * _NB3 + j, 0)),
          pl.BlockSpec((1, _H, _HH), lambda g, j: (g, 0, 0)),
          pl.BlockSpec((1, 1, _HH), lambda g, j: (g, 0, 0)),
          pl.BlockSpec((_H, _H4), lambda g, j: (0, 0)),
          pl.BlockSpec((1, _H4), lambda g, j: (0, 0)),
          pl.BlockSpec((_H4, 1), lambda g, j: (0, 0)),
          pl.BlockSpec((1, 1), lambda g, j: (0, 0)),
      ],
      out_specs=pl.BlockSpec((1, 1), lambda g, j: (0, 0)),
      out_shape=jax.ShapeDtypeStruct((1, 1), jnp.float32),
      scratch_shapes=[
          pltpu.VMEM((1, _HH), jnp.float32),
          pltpu.VMEM((1, _HH), jnp.float32),
      ],
  )(a_cat, c_cat, Wb, bb, Wd, bd, Wo, bo)


def kernel(x1, edge_index1, x2, edge_index2, W1a, b1a, W1b, b1b,
           W2a, b2a, W2b, b2b, Wd, bd, Wo, bo):
  x_cat = jnp.concatenate([x1, x2], axis=0)
  Wa = jnp.stack([W1a, W2a])
  ba = jnp.stack([b1a, b2a]).reshape(2, 1, _H)
  Wb = jnp.stack([W1b, W2b])
  bb = jnp.stack([b1b, b2b]).reshape(2, 1, _HH)

  src_off = jnp.concatenate([edge_index1[0], edge_index2[0] + _N])
  src_raw = jnp.concatenate([edge_index1[0], edge_index2[0]])
  dst_raw = jnp.concatenate([edge_index1[1], edge_index2[1]])
  nrow = 2 * _E // _K
  src_off = src_off.reshape(nrow, _K)
  src_raw = src_raw.reshape(nrow, _K)
  dst_raw = dst_raw.reshape(nrow, _K)

  z_h = jnp.zeros((_ZR, _H), jnp.float32)
  z_c = jnp.zeros((_ZR, _G), jnp.float32)
  ones_g = jnp.ones((_K, _G), jnp.float32)

  h_cat = _stage1(x_cat, Wa, ba)
  a_cat, c_cat = _sc_scatter(h_cat, src_off, src_raw, dst_raw, z_h, z_c,
                             ones_g)
  return _stage3(a_cat, c_cat, Wb, bb, Wd, bd.reshape(1, _H4),
                 Wo.reshape(_H4, 1), bo.reshape(1, 1))


# speedup vs baseline: 18.8739x; 1.6614x over previous
import functools

import jax
import jax.numpy as jnp
from jax import lax
from jax.experimental import pallas as pl
from jax.experimental.pallas import tpu as pltpu
from jax.experimental.pallas import tpu_sc as plsc

_N = 10000
_E = 320000
_D = 128
_H = 64
_HH = _H // 2
_H4 = _H // 4

_NS = 16
_K = 125
_NCH = _E // (_NS * _K)
_NP = 10240
_ZR = _NP // _NS
_G = 8

_NB1 = 5
_RB1 = _N // _NB1
_NB3 = 10
_RB3 = _NP // _NB3


def _stage1(x, W, b):
  def body(x_ref, w_ref, b_ref, o_ref):
    o_ref[...] = jnp.maximum(
        jnp.dot(x_ref[...], w_ref[...], preferred_element_type=jnp.float32)
        + b_ref[...], 0.0)

  return pl.pallas_call(
      body,
      grid=(_NB1,),
      in_specs=[
          pl.BlockSpec((_RB1, _D), lambda j: (j, 0)),
          pl.BlockSpec((_D, _H), lambda j: (0, 0)),
          pl.BlockSpec((1, _H), lambda j: (0, 0)),
      ],
      out_specs=pl.BlockSpec((_RB1, _H), lambda j: (j, 0)),
      out_shape=jax.ShapeDtypeStruct((_N, _H), jnp.float32),
  )(x, W, b)


def _sc_scatter(h1, src1, dst1, h2, src2, dst2, z_h, z_c, ones_g):
  mesh = plsc.VectorSubcoreMesh(core_axis_name="c", subcore_axis_name="s")

  @functools.partial(
      pl.kernel,
      out_type=[
          jax.ShapeDtypeStruct((2 * _NP, _H), jnp.float32),
          jax.ShapeDtypeStruct((2 * _NP, _G), jnp.float32),
      ],
      mesh=mesh,
      scratch_types=[
          pltpu.VMEM((_NCH, _K), jnp.int32),
          pltpu.VMEM((_NCH, _K), jnp.int32),
          pltpu.VMEM((_K, _H), jnp.float32),
          pltpu.VMEM((_K, _H), jnp.float32),
          pltpu.VMEM((_K, _G), jnp.float32),
          pltpu.VMEM_SHARED((_NP, _H), jnp.float32),
          pltpu.VMEM_SHARED((_NP, _G), jnp.float32),
          pltpu.SemaphoreType.DMA,
          pltpu.SemaphoreType.DMA,
      ],
      compiler_params=pltpu.CompilerParams(use_tc_tiling_on_sc=False),
  )
  def k(h1_hbm, s1_hbm, d1_hbm, h2_hbm, s2_hbm, d2_hbm,
        zh_hbm, zc_hbm, ones_hbm, a_out, c_out,
        src_v, dst_v, msg0, msg1, ones_v, a_s, c_s, sem0, sem1):
    cid = lax.axis_index("c")
    sid = lax.axis_index("s")
    pltpu.sync_copy(zh_hbm, a_s.at[pl.ds(sid * _ZR, _ZR)])
    pltpu.sync_copy(zc_hbm, c_s.at[pl.ds(sid * _ZR, _ZR)])
    pltpu.sync_copy(ones_hbm, ones_v)
    row0 = sid * _NCH

    @pl.when(cid == 0)
    def _():
      pltpu.sync_copy(s1_hbm.at[pl.ds(row0, _NCH)], src_v)
      pltpu.sync_copy(d1_hbm.at[pl.ds(row0, _NCH)], dst_v)

    @pl.when(cid == 1)
    def _():
      pltpu.sync_copy(s2_hbm.at[pl.ds(row0, _NCH)], src_v)
      pltpu.sync_copy(d2_hbm.at[pl.ds(row0, _NCH)], dst_v)

    plsc.subcore_barrier()

    def edge_loop(h_hbm):
      msgs = (msg0, msg1)
      sems = (sem0, sem1)
      pltpu.async_copy(h_hbm.at[src_v.at[0]], msg0, sem0)
      pltpu.async_copy(h_hbm.at[src_v.at[1]], msg1, sem1)

      @pl.loop(0, _NCH, step=2)
      def _(j):
        for b in range(2):
          jj = j + b
          pltpu.make_async_copy(h_hbm.at[src_v.at[jj]], msgs[b],
                                sems[b]).wait()
          pltpu.sync_copy(msgs[b], a_s.at[dst_v.at[jj]], add=True)
          pltpu.sync_copy(ones_v, c_s.at[src_v.at[jj]], add=True)

          @pl.when(jj + 2 < _NCH)
          def _():
            pltpu.async_copy(h_hbm.at[src_v.at[jj + 2]], msgs[b], sems[b])

    @pl.when(cid == 0)
    def _():
      edge_loop(h1_hbm)

    @pl.when(cid == 1)
    def _():
      edge_loop(h2_hbm)

    plsc.subcore_barrier()
    base = cid * _NP + sid * _ZR
    pltpu.sync_copy(a_s.at[pl.ds(sid * _ZR, _ZR)], a_out.at[pl.ds(base, _ZR)])
    pltpu.sync_copy(c_s.at[pl.ds(sid * _ZR, _ZR)], c_out.at[pl.ds(base, _ZR)])

  return k(h1, src1, dst1, h2, src2, dst2, z_h, z_c, ones_g)


def _stage3(a_cat, c_cat, Wb, bb, Wd, bd, Wo, bo):
  def body(a_ref, c_ref, wb, bbr, wd, bdr, wo, bor, o_ref, acc0, acc1):
    g = pl.program_id(0)
    j = pl.program_id(1)
    y = jnp.dot(a_ref[...], wb[0], preferred_element_type=jnp.float32) + bbr[0]
    y = jnp.maximum(y, 0.0)
    part = jnp.sum(y * c_ref[:, 0:1], axis=0, keepdims=True)

    @pl.when((g == 0) & (j == 0))
    def _():
      acc0[...] = part

    @pl.when((g == 0) & (j > 0))
    def _():
      acc0[...] = acc0[...] + part

    @pl.when((g == 1) & (j == 0))
    def _():
      acc1[...] = part

    @pl.when((g == 1) & (j > 0))
    def _():
      acc1[...] = acc1[...] + part

    @pl.when((g == 1) & (j == _NB3 - 1))
    def _():
      p0 = acc0[...] * (1.0 / _N)
      p1 = acc1[...] * (1.0 / _N)
      t = (jnp.dot(p0, wd[0:_HH], preferred_element_type=jnp.float32)
           + jnp.dot(p1, wd[_HH:_H], preferred_element_type=jnp.float32)
           + bdr[...])
      t = jnp.maximum(t, 0.0)
      z = jnp.dot(t, wo[...], preferred_element_type=jnp.float32) + bor[...]
      o_ref[...] = 1.0 / (1.0 + jnp.exp(-z))

  return pl.pallas_call(
      body,
      grid=(2, _NB3),
      in_specs=[
          pl.BlockSpec((_RB3, _H), lambda g, j: (g * _NB3 + j, 0)),
          pl.BlockSpec((_RB3, _G), lambda g, j: (g * _NB3 + j, 0)),
          pl.BlockSpec((1, _H, _HH), lambda g, j: (g, 0, 0)),
          pl.BlockSpec((1, 1, _HH), lambda g, j: (g, 0, 0)),
          pl.BlockSpec((_H, _H4), lambda g, j: (0, 0)),
          pl.BlockSpec((1, _H4), lambda g, j: (0, 0)),
          pl.BlockSpec((_H4, 1), lambda g, j: (0, 0)),
          pl.BlockSpec((1, 1), lambda g, j: (0, 0)),
      ],
      out_specs=pl.BlockSpec((1, 1), lambda g, j: (0, 0)),
      out_shape=jax.ShapeDtypeStruct((1, 1), jnp.float32),
      scratch_shapes=[
          pltpu.VMEM((1, _HH), jnp.float32),
          pltpu.VMEM((1, _HH), jnp.float32),
      ],
  )(a_cat, c_cat, Wb, bb, Wd, bd, Wo, bo)


def kernel(x1, edge_index1, x2, edge_index2, W1a, b1a, W1b, b1b,
           W2a, b2a, W2b, b2b, Wd, bd, Wo, bo):
  nrow = _E // _K
  src1 = edge_index1[0].reshape(nrow, _K)
  dst1 = edge_index1[1].reshape(nrow, _K)
  src2 = edge_index2[0].reshape(nrow, _K)
  dst2 = edge_index2[1].reshape(nrow, _K)

  z_h = jnp.zeros((_ZR, _H), jnp.float32)
  z_c = jnp.zeros((_ZR, _G), jnp.float32)
  ones_g = jnp.ones((_K, _G), jnp.float32)

  h1 = _stage1(x1, W1a, b1a.reshape(1, _H))
  h2 = _stage1(x2, W2a, b2a.reshape(1, _H))
  a_cat, c_cat = _sc_scatter(h1, src1, dst1, h2, src2, dst2, z_h, z_c,
                             ones_g)

  Wb = jnp.stack([W1b, W2b])
  bb = jnp.stack([b1b, b2b]).reshape(2, 1, _HH)
  return _stage3(a_cat, c_cat, Wb, bb, Wd, bd.reshape(1, _H4),
                 Wo.reshape(_H4, 1), bo.reshape(1, 1))
